# baseline (speedup 1.0000x reference)
import functools

import jax
import jax.numpy as jnp
from jax import lax
from jax.experimental import pallas as pl
from jax.experimental.pallas import tpu as pltpu
from jax.experimental.pallas import tpu_sc as plsc

N_ENTRIES = 1000000
PARAM_DIM = 64
BATCH = 16384
PAD = 2 * PARAM_DIM

NC = 2
NS = 16
NW = NC * NS
L = 16

R_W = 31264
CH = 400
N_CH = 80
LIST_CAP = 1024
SLOTS = 64
XBLK = 1024
G_SCAN = XBLK // L

_mesh = plsc.VectorSubcoreMesh(core_axis_name="c", subcore_axis_name="s")


@functools.partial(
    pl.kernel,
    out_type=jax.ShapeDtypeStruct((BATCH + NW, PAD), jnp.float32),
    mesh=_mesh,
    scratch_types=[
        pltpu.VMEM((XBLK,), jnp.int32),
        pltpu.VMEM((LIST_CAP + L,), jnp.int32),
        pltpu.VMEM((LIST_CAP + L,), jnp.int32),
        pltpu.VMEM((CH, PARAM_DIM), jnp.float32),
        pltpu.VMEM((CH, PARAM_DIM), jnp.float32),
        pltpu.VMEM((SLOTS,), jnp.int32),
        pltpu.VMEM((SLOTS,), jnp.int32),
        pltpu.VMEM((SLOTS, PAD), jnp.float32),
        pltpu.SemaphoreType.DMA,
        pltpu.SemaphoreType.DMA,
        pltpu.SemaphoreType.DMA,
    ],
    compiler_params=pltpu.CompilerParams(needs_layout_passes=False),
)
def _lookup_kernel(x_hbm, table_hbm, out_hbm, idx_v, plist, rlist, buf0,
                   buf1, cpos0, clrow, stg0, sem_a, sem_b, sem_s0):
    wid = lax.axis_index("s") * NC + lax.axis_index("c")
    lo = wid * R_W
    hi = lo + R_W
    trash = BATCH + wid

    cnt = jnp.int32(0)
    for b in range(BATCH // XBLK):
        pltpu.sync_copy(x_hbm.at[pl.ds(b * XBLK, XBLK)], idx_v)

        def scan(g, cnt, _b=b):
            v = idx_v[pl.ds(g * L, L)]
            m = jax.lax.lt(v, jnp.int32(hi)) & jax.lax.ge(v, jnp.int32(lo))
            pos = jax.lax.iota(jnp.int32, L) + (_b * XBLK + g * L)
            plsc.store_compressed(plist.at[pl.ds(cnt, L)], pos, mask=m)
            plsc.store_compressed(rlist.at[pl.ds(cnt, L)], v, mask=m)
            n16 = plsc.all_reduce_population_count(m)
            return jnp.minimum(cnt + n16[0], LIST_CAP)

        cnt = lax.fori_loop(0, G_SCAN, scan, cnt)
    n_scan_grp = jax.lax.div(cnt + (L - 1), L)

    bufs = (buf0, buf1)
    cposs = (cpos0, cpos0)
    rd_sems = (sem_a, sem_b)

    def chunk_start(c):
        return jnp.minimum(
            jnp.int32(lo) + jnp.int32(CH) * c, jnp.int32(N_ENTRIES - CH)
        )

    def fire_read(c, b):
        cst = pl.multiple_of(chunk_start(c), 8)
        pltpu.async_copy(
            table_hbm.at[pl.ds(cst, CH), :], bufs[b], rd_sems[b]
        )

    for b in range(2):
        for j in range(SLOTS // L):
            cposs[b][pl.ds(j * L, L)] = jnp.full((L,), trash, jnp.int32)
    pltpu.async_copy(stg0, out_hbm.at[cpos0], sem_s0)
    fire_read(jnp.int32(0), 0)

    def process(c, b):
        pltpu.make_async_copy(
            table_hbm.at[pl.ds(0, CH), :], bufs[b], rd_sems[b]
        ).wait()
        pltpu.make_async_copy(
            stg0, out_hbm.at[pl.ds(0, SLOTS)], sem_s0
        ).wait()
        cst = chunk_start(c)

        for j in range(SLOTS // L):
            cpos0[pl.ds(j * L, L)] = jnp.full((L,), trash, jnp.int32)

        def filt(j, ccnt):
            lane = jax.lax.iota(jnp.int32, L) + j * L
            p16 = plist[pl.ds(j * L, L)]
            r16 = rlist[pl.ds(j * L, L)]
            ok = jax.lax.lt(lane, cnt)
            ok = ok & jax.lax.ge(r16, cst) & jax.lax.lt(r16, cst + CH)
            plsc.store_compressed(cpos0.at[pl.ds(ccnt, L)], p16, mask=ok)
            plsc.store_compressed(clrow.at[pl.ds(ccnt, L)], r16 - cst, mask=ok)
            n16 = plsc.all_reduce_population_count(ok)
            return jnp.minimum(ccnt + n16[0], SLOTS - L)

        ccnt = lax.fori_loop(0, n_scan_grp, filt, jnp.int32(0))

        def stage(g2, _):
            e16 = jax.lax.iota(jnp.int32, L) + g2 * L
            lane_ok = jax.lax.lt(e16, ccnt)
            lr16 = jnp.where(lane_ok, clrow[pl.ds(g2 * L, L)], 0)
            for cc in range(PARAM_DIM):
                col = jnp.full((L,), cc, jnp.int32)
                vals = plsc.load_gather(bufs[b], [lr16, col])
                plsc.store_scatter(stg0, [e16, col], vals)
            return 0

        lax.fori_loop(0, jax.lax.div(ccnt + (L - 1), L), stage, 0)

        pltpu.async_copy(stg0, out_hbm.at[cpos0], sem_s0)

    def pair(cc, _):
        c0 = cc * 2
        fire_read(c0 + 1, 1)
        process(c0, 0)
        fire_read(c0 + 2, 0)
        process(c0 + 1, 1)
        return 0

    lax.fori_loop(0, N_CH // 2, pair, 0)

    pltpu.make_async_copy(
        table_hbm.at[pl.ds(0, CH), :], buf0, sem_a
    ).wait()
    pltpu.make_async_copy(
        stg0, out_hbm.at[pl.ds(0, SLOTS)], sem_s0
    ).wait()


def kernel(x, table):
    y = _lookup_kernel(x, table)
    return y[:BATCH, :PARAM_DIM]


# speedup vs baseline: 2.4465x; 2.4465x over previous
import functools

import jax
import jax.numpy as jnp
from jax import lax
from jax.experimental import pallas as pl
from jax.experimental.pallas import tpu as pltpu
from jax.experimental.pallas import tpu_sc as plsc

N_ENTRIES = 1000000
PARAM_DIM = 64
BATCH = 16384

NC = 2
NS = 16
NW = NC * NS
B_PER_W = BATCH // NW
L = 16
NSEM = 8

_mesh = plsc.VectorSubcoreMesh(core_axis_name="c", subcore_axis_name="s")


@functools.partial(
    pl.kernel,
    out_type=jax.ShapeDtypeStruct((BATCH, PARAM_DIM), jnp.float32),
    mesh=_mesh,
    scratch_types=[
        pltpu.VMEM((B_PER_W,), jnp.int32),
        pltpu.VMEM((B_PER_W, PARAM_DIM), jnp.float32),
    ] + [pltpu.SemaphoreType.DMA] * NSEM,
    compiler_params=pltpu.CompilerParams(needs_layout_passes=False),
)
def _lookup_kernel(x_hbm, table_hbm, out_hbm, idx_v, out_v, *sems):
    wid = lax.axis_index("s") * NC + lax.axis_index("c")
    base = wid * B_PER_W

    pltpu.sync_copy(x_hbm.at[pl.ds(base, B_PER_W)], idx_v)

    def body(g, _):
        vec = idx_v[pl.ds(g * L, L)]
        for k2 in range(L):
            i = vec[k2]
            pltpu.async_copy(
                table_hbm.at[i], out_v.at[g * L + k2], sems[k2 % NSEM],
            )
        return 0

    lax.fori_loop(0, B_PER_W // L, body, 0)
    for q in range(NSEM):
        pltpu.make_async_copy(
            table_hbm.at[pl.ds(0, B_PER_W // NSEM)],
            out_v.reshape(NSEM, B_PER_W // NSEM, PARAM_DIM).at[q],
            sems[q],
        ).wait()

    pltpu.sync_copy(out_v, out_hbm.at[pl.ds(base, B_PER_W)])


def kernel(x, table):
    return _lookup_kernel(x, table)
